# baseline (speedup 1.0000x reference)
import functools

import jax
import jax.numpy as jnp
from jax.experimental import pallas as pl
from jax.experimental.pallas import tpu as pltpu

MAX_DIST = 32
TABLE_SIZE = 2 * MAX_DIST + 1
VALS = MAX_DIST + 1
K = 128
NBUF = 4


def _bias_kernel(o_all_ref, m_ref, out_ref, scr_a, scr_b, sem_a, sem_b,
                 *, nh, nsteps):
    b = pl.program_id(0)
    h = pl.program_id(1)
    step = b * nh + h
    parity = jax.lax.rem(step, 2)
    slot = jax.lax.rem(step // 2, NBUF // 2)

    of = o_all_ref[0]
    m = m_ref[0].astype(jnp.bfloat16)
    a = jnp.dot(of, m, preferred_element_type=jnp.float32)
    out = jax.lax.dot_general(
        a.astype(jnp.bfloat16), of, (((1,), (1,)), ((), ())),
        preferred_element_type=jnp.float32)

    @pl.when(parity == 0)
    def _even():
        @pl.when(step >= NBUF)
        def _wait_prev():
            pltpu.make_async_copy(
                scr_a.at[slot], out_ref.at[b, h], sem_a.at[slot]).wait()
        scr_a[slot] = out
        pltpu.make_async_copy(
            scr_a.at[slot], out_ref.at[b, h], sem_a.at[slot]).start()

    @pl.when(parity == 1)
    def _odd():
        @pl.when(step >= NBUF)
        def _wait_prev():
            pltpu.make_async_copy(
                scr_b.at[slot], out_ref.at[b, h], sem_b.at[slot]).wait()
        scr_b[slot] = out
        pltpu.make_async_copy(
            scr_b.at[slot], out_ref.at[b, h], sem_b.at[slot]).start()

    @pl.when(step == nsteps - 1)
    def _drain():
        for k in range(NBUF):
            so = nsteps - NBUF + k
            sl = (so // 2) % (NBUF // 2)
            scr = scr_a if so % 2 == 0 else scr_b
            sem = sem_a if so % 2 == 0 else sem_b
            pltpu.make_async_copy(
                scr.at[sl], out_ref.at[so // nh, so % nh], sem.at[sl]).wait()


@functools.partial(jax.jit, static_argnames=())
def kernel(positions, rel_bias_d, rel_bias_h, rel_bias_w):
    B, N, _ = positions.shape
    H = rel_bias_d.shape[1]

    pos = jnp.clip(positions.astype(jnp.int32), 0, MAX_DIST)
    ks = jnp.arange(K, dtype=jnp.int32)
    onehot = ((pos[:, :, 0, None] == ks)
              | (pos[:, :, 1, None] + VALS == ks)
              | (pos[:, :, 2, None] + 2 * VALS == ks)).astype(jnp.bfloat16)

    u = jnp.arange(VALS, dtype=jnp.int32)
    duv = u[:, None] - u[None, :] + MAX_DIST
    md = rel_bias_d[duv].transpose(2, 0, 1)
    mh = rel_bias_h[duv].transpose(2, 0, 1)
    mw = rel_bias_w[duv].transpose(2, 0, 1)
    m = jnp.zeros((H, K, K), dtype=jnp.float32)
    m = m.at[:, 0:VALS, 0:VALS].set(md)
    m = m.at[:, VALS:2 * VALS, VALS:2 * VALS].set(mh)
    m = m.at[:, 2 * VALS:3 * VALS, 2 * VALS:3 * VALS].set(mw)

    grid = (B, H)
    out = pl.pallas_call(
        functools.partial(_bias_kernel, nh=H, nsteps=B * H),
        grid=grid,
        in_specs=[
            pl.BlockSpec((1, N, K), lambda b, h: (b, 0, 0)),
            pl.BlockSpec((1, K, K), lambda b, h: (h, 0, 0)),
        ],
        out_specs=pl.BlockSpec(memory_space=pl.ANY),
        out_shape=jax.ShapeDtypeStruct((B, H, N, N), jnp.float32),
        scratch_shapes=[
            pltpu.VMEM((NBUF // 2, N, N), jnp.float32),
            pltpu.VMEM((NBUF // 2, N, N), jnp.float32),
            pltpu.SemaphoreType.DMA((NBUF // 2,)),
            pltpu.SemaphoreType.DMA((NBUF // 2,)),
        ],
    )(onehot, m)
    return out


# speedup vs baseline: 1.0203x; 1.0203x over previous
import functools

import jax
import jax.numpy as jnp
from jax.experimental import pallas as pl
from jax.experimental.pallas import tpu as pltpu

MAX_DIST = 32
TABLE_SIZE = 2 * MAX_DIST + 1
VALS = MAX_DIST + 1
K = 128
NSLOT = 2


def _bias_kernel(o_all_ref, m_ref, out_ref, scr_a, scr_b, sem_a, sem_b,
                 *, nh, nsteps):
    b = pl.program_id(0)
    g = pl.program_id(1)
    step = b * (nh // 2) + g
    slot = jax.lax.rem(step, NSLOT)
    hlo = g
    hhi = g + nh // 2

    of = o_all_ref[0]

    def one_head(mm):
        a = jnp.dot(of, mm.astype(jnp.bfloat16),
                    preferred_element_type=jnp.float32)
        return jax.lax.dot_general(
            a.astype(jnp.bfloat16), of, (((1,), (1,)), ((), ())),
            preferred_element_type=jnp.float32)

    @pl.when(step >= NSLOT)
    def _wait_prev():
        pltpu.make_async_copy(
            scr_a.at[slot], out_ref.at[b, hlo], sem_a.at[slot]).wait()
        pltpu.make_async_copy(
            scr_b.at[slot], out_ref.at[b, hhi], sem_b.at[slot]).wait()

    scr_a[slot] = one_head(m_ref[0, 0])
    pltpu.make_async_copy(
        scr_a.at[slot], out_ref.at[b, hlo], sem_a.at[slot]).start()
    scr_b[slot] = one_head(m_ref[0, 1])
    pltpu.make_async_copy(
        scr_b.at[slot], out_ref.at[b, hhi], sem_b.at[slot]).start()

    @pl.when(step == nsteps - 1)
    def _drain():
        for k in range(NSLOT):
            so = nsteps - NSLOT + k
            sl = so % NSLOT
            bo = so // (nh // 2)
            go = so % (nh // 2)
            pltpu.make_async_copy(
                scr_a.at[sl], out_ref.at[bo, go], sem_a.at[sl]).wait()
            pltpu.make_async_copy(
                scr_b.at[sl], out_ref.at[bo, go + nh // 2],
                sem_b.at[sl]).wait()


@functools.partial(jax.jit, static_argnames=())
def kernel(positions, rel_bias_d, rel_bias_h, rel_bias_w):
    B, N, _ = positions.shape
    H = rel_bias_d.shape[1]

    pos = jnp.clip(positions.astype(jnp.int32), 0, MAX_DIST)
    ks = jnp.arange(K, dtype=jnp.int32)
    onehot = ((pos[:, :, 0, None] == ks)
              | (pos[:, :, 1, None] + VALS == ks)
              | (pos[:, :, 2, None] + 2 * VALS == ks)).astype(jnp.bfloat16)

    u = jnp.arange(VALS, dtype=jnp.int32)
    duv = u[:, None] - u[None, :] + MAX_DIST
    md = rel_bias_d[duv].transpose(2, 0, 1)
    mh = rel_bias_h[duv].transpose(2, 0, 1)
    mw = rel_bias_w[duv].transpose(2, 0, 1)
    m = jnp.zeros((H, K, K), dtype=jnp.float32)
    m = m.at[:, 0:VALS, 0:VALS].set(md)
    m = m.at[:, VALS:2 * VALS, VALS:2 * VALS].set(mh)
    m = m.at[:, 2 * VALS:3 * VALS, 2 * VALS:3 * VALS].set(mw)

    m_pairs = jnp.stack([m[: H // 2], m[H // 2:]], axis=1)

    grid = (B, H // 2)
    out = pl.pallas_call(
        functools.partial(_bias_kernel, nh=H, nsteps=B * (H // 2)),
        grid=grid,
        in_specs=[
            pl.BlockSpec((1, N, K), lambda b, g: (b, 0, 0)),
            pl.BlockSpec((1, 2, K, K), lambda b, g: (g, 0, 0, 0)),
        ],
        out_specs=pl.BlockSpec(memory_space=pl.ANY),
        out_shape=jax.ShapeDtypeStruct((B, H, N, N), jnp.float32),
        scratch_shapes=[
            pltpu.VMEM((NSLOT, N, N), jnp.float32),
            pltpu.VMEM((NSLOT, N, N), jnp.float32),
            pltpu.SemaphoreType.DMA((NSLOT,)),
            pltpu.SemaphoreType.DMA((NSLOT,)),
        ],
    )(onehot, m_pairs)
    return out
